# baseline (speedup 1.0000x reference)
import functools

import jax
import jax.numpy as jnp
from jax import lax
from jax.experimental import pallas as pl
from jax.experimental.pallas import tpu as pltpu
from jax.experimental.pallas import tpu_sc as plsc

N, D, H, E, K = 32768, 768, 64, 64, 2

SC_ROWS = 12288
TC_ROWS = N - SC_ROWS

TC_CHUNK = 2048
TC_GRID = TC_ROWS // TC_CHUNK

NC, NS, L = 2, 16, 16
NW = NC * NS
RPW = SC_ROWS // NW
R = 64
NCH = RPW // R
NV = D // L


def _tc_partial_body(x_ref, w_ref, out_ref, acc_ref):
    i = pl.program_id(0)

    @pl.when(i == 0)
    def _init():
        acc_ref[...] = jnp.zeros_like(acc_ref)

    acc_ref[...] += jax.lax.dot_general(
        w_ref[...], x_ref[...], (((1,), (0,)), ((), ())),
        preferred_element_type=jnp.float32)

    @pl.when(i == TC_GRID - 1)
    def _finish():
        out_ref[...] = acc_ref[...]


def _sc_partial_body(x_hbm, w_hbm, out_hbm, xbuf, wbuf, accv, xsem, wsem):
    c = lax.axis_index("c")
    s = lax.axis_index("s")
    wid = s * NC + c
    base = TC_ROWS + wid * RPW

    pltpu.make_async_copy(
        w_hbm.at[pl.ds(base, RPW)], wbuf.at[pl.ds(0, RPW)], wsem).start()

    def chunk_copy(k):
        return pltpu.make_async_copy(
            x_hbm.at[pl.ds(base + k * R, R), :], xbuf.at[k % 2],
            xsem.at[k % 2])

    chunk_copy(0).start()
    chunk_copy(1).start()
    pltpu.make_async_copy(
        w_hbm.at[pl.ds(base, RPW)], wbuf.at[pl.ds(0, RPW)], wsem).wait()

    regs = tuple(jnp.zeros((L,), jnp.float32) for _ in range(NV))
    for k in range(NCH):
        chunk_copy(k).wait()
        slot = k % 2

        def row_body(r, rg, k=k, slot=slot):
            w = wbuf[pl.ds(k * R + r, L)][0]
            return tuple(
                rg[j] + xbuf[slot, r, pl.ds(j * L, L)] * w
                for j in range(NV))

        regs = lax.fori_loop(0, R, row_body, regs)
        if k + 2 < NCH:
            chunk_copy(k + 2).start()

    for j in range(NV):
        accv[pl.ds(j * L, L)] = regs[j]
    pltpu.sync_copy(accv, out_hbm.at[wid])


def _gate_body(ptc_ref, psc_ref, win_ref, wlin_ref, idx_ref, p_ref):
    v = ptc_ref[...] + jnp.sum(psc_ref[...], axis=0, keepdims=True)
    h = jax.lax.dot_general(
        v, win_ref[...], (((1,), (1,)), ((), ())),
        preferred_element_type=jnp.float32)
    o = jax.lax.dot_general(
        h, wlin_ref[...], (((1,), (1,)), ((), ())),
        preferred_element_type=jnp.float32)

    iota = jax.lax.broadcasted_iota(jnp.int32, (1, E), 1)
    m1 = jnp.max(o)
    i1 = jnp.min(jnp.where(o == m1, iota, E))
    masked = jnp.where(iota == i1, -jnp.inf, o)
    m2 = jnp.max(masked)
    i2 = jnp.min(jnp.where(masked == m2, iota, E))
    e = jnp.exp(m2 - m1)
    p1 = 1.0 / (1.0 + e)

    pos = jax.lax.broadcasted_iota(jnp.int32, (1, 2), 1)
    idx_ref[...] = jnp.where(pos == 0, i1, i2)
    p_ref[...] = jnp.where(pos == 0, p1, 1.0 - p1)


_sc_partial = functools.partial(
    pl.kernel,
    out_type=jax.ShapeDtypeStruct((NW, D), jnp.float32),
    mesh=plsc.VectorSubcoreMesh(core_axis_name="c", subcore_axis_name="s"),
    scratch_types=[
        pltpu.VMEM((2, R, D), jnp.float32),
        pltpu.VMEM((RPW + L,), jnp.float32),
        pltpu.VMEM((D,), jnp.float32),
        pltpu.SemaphoreType.DMA((2,)),
        pltpu.SemaphoreType.DMA,
    ],
)(_sc_partial_body)


@jax.jit
def kernel(x, W_in, W_lin, W_out):
    ptc = pl.pallas_call(
        _tc_partial_body,
        grid=(TC_GRID,),
        in_specs=[
            pl.BlockSpec((TC_CHUNK, D), lambda i: (i, 0)),
            pl.BlockSpec((1, TC_CHUNK), lambda i: (0, i)),
        ],
        out_specs=pl.BlockSpec((1, D), lambda i: (0, 0)),
        out_shape=jax.ShapeDtypeStruct((1, D), jnp.float32),
        scratch_shapes=[pltpu.VMEM((1, D), jnp.float32)],
    )(x, W_out)

    psc = _sc_partial(x, W_out.reshape(-1))

    idx2, p2 = pl.pallas_call(
        _gate_body,
        out_shape=[
            jax.ShapeDtypeStruct((1, 2), jnp.int32),
            jax.ShapeDtypeStruct((1, 2), jnp.float32),
        ],
    )(ptc, psc, W_in, W_lin)
    return idx2.reshape(-1), p2.reshape(-1)


# speedup vs baseline: 1.5650x; 1.5650x over previous
import functools

import jax
import jax.numpy as jnp
from jax.experimental import pallas as pl
from jax.experimental.pallas import tpu as pltpu

N, D, H, E, K = 32768, 768, 64, 64, 2
CHUNK = 4096
GRID = N // CHUNK


def _gate_body(x_ref, w_ref, win_ref, wlin_ref, idx_ref, p_ref, acc_ref):
    i = pl.program_id(0)

    @pl.when(i == 0)
    def _init():
        acc_ref[...] = jnp.zeros_like(acc_ref)

    acc_ref[...] += jax.lax.dot_general(
        w_ref[...], x_ref[...], (((1,), (0,)), ((), ())),
        preferred_element_type=jnp.float32)

    @pl.when(i == GRID - 1)
    def _finish():
        v = acc_ref[...]
        h = jax.lax.dot_general(
            v, win_ref[...], (((1,), (1,)), ((), ())),
            preferred_element_type=jnp.float32)
        o = jax.lax.dot_general(
            h, wlin_ref[...], (((1,), (1,)), ((), ())),
            preferred_element_type=jnp.float32)

        iota = jax.lax.broadcasted_iota(jnp.int32, (1, E), 1)
        m1 = jnp.max(o)
        i1 = jnp.min(jnp.where(o == m1, iota, E))
        masked = jnp.where(iota == i1, -jnp.inf, o)
        m2 = jnp.max(masked)
        i2 = jnp.min(jnp.where(masked == m2, iota, E))
        e = jnp.exp(m2 - m1)
        p1 = 1.0 / (1.0 + e)

        pos = jax.lax.broadcasted_iota(jnp.int32, (1, 2), 1)
        idx_ref[...] = jnp.where(pos == 0, i1, i2)
        p_ref[...] = jnp.where(pos == 0, p1, 1.0 - p1)


@functools.partial(jax.jit, static_argnames=("interpret",))
def kernel(x, W_in, W_lin, W_out, interpret=False):
    idx2, p2 = pl.pallas_call(
        _gate_body,
        grid=(GRID,),
        in_specs=[
            pl.BlockSpec((CHUNK, D), lambda i: (i, 0)),
            pl.BlockSpec((1, CHUNK), lambda i: (0, i)),
            pl.BlockSpec((H, D), lambda i: (0, 0)),
            pl.BlockSpec((E, H), lambda i: (0, 0)),
        ],
        out_specs=[
            pl.BlockSpec((1, 2), lambda i: (0, 0)),
            pl.BlockSpec((1, 2), lambda i: (0, 0)),
        ],
        out_shape=[
            jax.ShapeDtypeStruct((1, 2), jnp.int32),
            jax.ShapeDtypeStruct((1, 2), jnp.float32),
        ],
        scratch_shapes=[pltpu.VMEM((1, D), jnp.float32)],
        interpret=interpret,
    )(x, W_out, W_in, W_lin)
    return idx2.reshape(-1), p2.reshape(-1)


# speedup vs baseline: 1.6422x; 1.0493x over previous
import functools

import jax
import jax.numpy as jnp
from jax.experimental import pallas as pl
from jax.experimental.pallas import tpu as pltpu

N, D, H, E, K = 32768, 768, 64, 64, 2
CHUNK = 2048
GRID = N // CHUNK


def _gate_body(x_ref, w_ref, win_ref, wlin_ref, idx_ref, p_ref, acc_ref):
    i = pl.program_id(0)

    @pl.when(i == 0)
    def _init():
        acc_ref[...] = jnp.zeros_like(acc_ref)

    acc_ref[...] += jax.lax.dot_general(
        w_ref[...], x_ref[...], (((1,), (0,)), ((), ())),
        preferred_element_type=jnp.float32)

    @pl.when(i == GRID - 1)
    def _finish():
        v = acc_ref[...]
        h = jax.lax.dot_general(
            v, win_ref[...], (((1,), (1,)), ((), ())),
            preferred_element_type=jnp.float32)
        o = jax.lax.dot_general(
            h, wlin_ref[...], (((1,), (1,)), ((), ())),
            preferred_element_type=jnp.float32)

        iota = jax.lax.broadcasted_iota(jnp.int32, (1, E), 1)
        m1 = jnp.max(o)
        i1 = jnp.min(jnp.where(o == m1, iota, E))
        masked = jnp.where(iota == i1, -jnp.inf, o)
        m2 = jnp.max(masked)
        i2 = jnp.min(jnp.where(masked == m2, iota, E))
        e = jnp.exp(m2 - m1)
        p1 = 1.0 / (1.0 + e)

        pos = jax.lax.broadcasted_iota(jnp.int32, (1, 2), 1)
        idx_ref[...] = jnp.where(pos == 0, i1, i2)
        p_ref[...] = jnp.where(pos == 0, p1, 1.0 - p1)


@functools.partial(jax.jit, static_argnames=("interpret",))
def kernel(x, W_in, W_lin, W_out, interpret=False):
    idx2, p2 = pl.pallas_call(
        _gate_body,
        grid=(GRID,),
        in_specs=[
            pl.BlockSpec((CHUNK, D), lambda i: (i, 0)),
            pl.BlockSpec((1, CHUNK), lambda i: (0, i)),
            pl.BlockSpec((H, D), lambda i: (0, 0)),
            pl.BlockSpec((E, H), lambda i: (0, 0)),
        ],
        out_specs=[
            pl.BlockSpec((1, 2), lambda i: (0, 0)),
            pl.BlockSpec((1, 2), lambda i: (0, 0)),
        ],
        out_shape=[
            jax.ShapeDtypeStruct((1, 2), jnp.int32),
            jax.ShapeDtypeStruct((1, 2), jnp.float32),
        ],
        scratch_shapes=[pltpu.VMEM((1, D), jnp.float32)],
        interpret=interpret,
    )(x, W_out, W_in, W_lin)
    return idx2.reshape(-1), p2.reshape(-1)
